# baseline (speedup 1.0000x reference)
import jax
import jax.numpy as jnp
from jax.experimental import pallas as pl
from jax.experimental.pallas import tpu as pltpu

_B, _I, _O, _K = 2048, 64, 64, 64
_N = _I * _K
_BT = 256


def _body(x_ref, c_ref, cb_ref, kn_ref, bias_ref, out_ref, hd_ref):
    b = pl.program_id(0)

    @pl.when(b == 0)
    def _slopes():
        y = c_ref[...]
        kn = kn_ref[...]
        hb = kn[:, 1:2] - kn[:, 0:1]
        inv_h = 1.0 / (hb + 1e-12)
        ki = jax.lax.broadcasted_iota(jnp.int32, (1, _N), 1) & (_K - 1)
        z1 = jnp.zeros((_O, 1), jnp.float32)
        z2 = jnp.zeros((_O, 2), jnp.float32)
        dl = jnp.concatenate([(y[:, 1:] - y[:, :-1]), z1], axis=1) * inv_h
        dm1 = jnp.concatenate([z1, dl[:, :-1]], axis=1)
        dm2 = jnp.concatenate([z2, dl[:, :-2]], axis=1)
        dp1 = jnp.concatenate([dl[:, 1:], z1], axis=1)

        w12 = 3.0 * hb
        same = dm1 * dl > 0
        dint = (w12 + w12) / (w12 / (dm1 + 1e-12) + w12 / (dl + 1e-12))
        dint = jnp.where(same, dint, jnp.zeros_like(dint))

        def limit(di, deltai):
            di = jnp.where(di * deltai <= 0, jnp.zeros_like(di), di)
            return jnp.where(jnp.abs(di) > 3.0 * jnp.abs(deltai),
                             3.0 * deltai, di)

        f0 = hb / (2.0 * hb + 1e-12)
        d0 = limit((3.0 * dl - dp1) * f0, dl)
        dN = limit((3.0 * dm1 - dm2) * f0, dm1)
        d = jnp.where(ki == 0, d0, jnp.where(ki == (_K - 1), dN, dint))
        hd_ref[...] = (d * hb).astype(jnp.bfloat16)

    x = x_ref[...]
    u = jnp.clip(x, 0.0, 1.0) * (_K - 1.0)
    col = jax.lax.broadcasted_iota(jnp.int32, (_I, _N), 1)
    row = jax.lax.broadcasted_iota(jnp.int32, (_I, _N), 0)
    rep = ((col >> 6) == row).astype(jnp.float32)
    dn0 = (((1,), (0,)), ((), ()))
    u_hi = u.astype(jnp.bfloat16).astype(jnp.float32)
    u_lo = u - u_hi
    urep = (jax.lax.dot_general(u_hi, rep, dn0,
                                preferred_element_type=jnp.float32)
            + jax.lax.dot_general(u_lo, rep, dn0,
                                  preferred_element_type=jnp.float32))
    kk = (jax.lax.broadcasted_iota(jnp.int32, (_BT, _N), 1)
          & (_K - 1)).astype(jnp.float32)
    s = urep - kk
    r = jnp.minimum(jnp.abs(s), 1.0)
    q = (1.0 - r) * (1.0 - r)
    wc = q * (1.0 + 2.0 * r)
    wd = q * s
    dn = (((1,), (1,)), ((), ()))
    acc = jax.lax.dot_general(wc.astype(jnp.bfloat16), cb_ref[...], dn,
                              preferred_element_type=jnp.float32)
    acc = acc + jax.lax.dot_general(wd.astype(jnp.bfloat16), hd_ref[...], dn,
                                    preferred_element_type=jnp.float32)
    out_ref[...] = acc + bias_ref[...]


def kernel(x, coeffs, bias, knots):
    c2 = coeffs.reshape(_O, _N)
    grid = _B // _BT
    out = pl.pallas_call(
        _body,
        grid=(grid,),
        in_specs=[
            pl.BlockSpec((_BT, _I), lambda b: (b, 0)),
            pl.BlockSpec((_O, _N), lambda b: (0, 0)),
            pl.BlockSpec((_O, _N), lambda b: (0, 0)),
            pl.BlockSpec((1, _K), lambda b: (0, 0)),
            pl.BlockSpec((1, _O), lambda b: (0, 0)),
        ],
        out_specs=pl.BlockSpec((_BT, _O), lambda b: (b, 0)),
        out_shape=jax.ShapeDtypeStruct((_B, _O), jnp.float32),
        scratch_shapes=[pltpu.VMEM((_O, _N), jnp.bfloat16)],
    )(x, c2, c2.astype(jnp.bfloat16), knots.reshape(1, _K),
      bias.reshape(1, _O))
    return out


# speedup vs baseline: 1.0294x; 1.0294x over previous
import jax
import jax.numpy as jnp
from jax.experimental import pallas as pl
from jax.experimental.pallas import tpu as pltpu

_B, _I, _O, _K = 2048, 64, 64, 64
_N = _I * _K
_BT = 512


def _body(x_ref, c_ref, kn_ref, bias_ref, out_ref, hd_ref):
    b = pl.program_id(0)

    @pl.when(b == 0)
    def _slopes():
        y = c_ref[...]
        kn = kn_ref[...]
        hb = kn[:, 1:2] - kn[:, 0:1]
        inv_h = 1.0 / (hb + 1e-12)
        ki = jax.lax.broadcasted_iota(jnp.int32, (1, _N), 1) & (_K - 1)
        z1 = jnp.zeros((_O, 1), jnp.float32)
        z2 = jnp.zeros((_O, 2), jnp.float32)
        dl = jnp.concatenate([(y[:, 1:] - y[:, :-1]), z1], axis=1) * inv_h
        dm1 = jnp.concatenate([z1, dl[:, :-1]], axis=1)
        dm2 = jnp.concatenate([z2, dl[:, :-2]], axis=1)
        dp1 = jnp.concatenate([dl[:, 1:], z1], axis=1)

        w12 = 3.0 * hb
        same = dm1 * dl > 0
        dint = (w12 + w12) / (w12 / (dm1 + 1e-12) + w12 / (dl + 1e-12))
        dint = jnp.where(same, dint, jnp.zeros_like(dint))

        def limit(di, deltai):
            di = jnp.where(di * deltai <= 0, jnp.zeros_like(di), di)
            return jnp.where(jnp.abs(di) > 3.0 * jnp.abs(deltai),
                             3.0 * deltai, di)

        f0 = hb / (2.0 * hb + 1e-12)
        d0 = limit((3.0 * dl - dp1) * f0, dl)
        dN = limit((3.0 * dm1 - dm2) * f0, dm1)
        d = jnp.where(ki == 0, d0, jnp.where(ki == (_K - 1), dN, dint))
        hd_ref[...] = d * hb

    x = x_ref[...]
    u = jnp.clip(x, 0.0, 1.0) * (_K - 1.0)
    col = jax.lax.broadcasted_iota(jnp.int32, (_I, _N), 1)
    row = jax.lax.broadcasted_iota(jnp.int32, (_I, _N), 0)
    rep = ((col >> 6) == row).astype(jnp.float32)
    dn0 = (((1,), (0,)), ((), ()))
    u_hi = u.astype(jnp.bfloat16).astype(jnp.float32)
    u_lo = u - u_hi
    urep = (jax.lax.dot_general(u_hi, rep, dn0,
                                preferred_element_type=jnp.float32)
            + jax.lax.dot_general(u_lo, rep, dn0,
                                  preferred_element_type=jnp.float32))
    kk = (jax.lax.broadcasted_iota(jnp.int32, (_BT, _N), 1)
          & (_K - 1)).astype(jnp.float32)
    s = urep - kk
    r = jnp.minimum(jnp.abs(s), 1.0)
    q = (1.0 - r) * (1.0 - r)
    wc = q * (1.0 + 2.0 * r)
    wd = q * s
    dn = (((1,), (1,)), ((), ()))
    acc = jax.lax.dot_general(wc, c_ref[...], dn,
                              preferred_element_type=jnp.float32)
    acc = acc + jax.lax.dot_general(wd, hd_ref[...], dn,
                                    preferred_element_type=jnp.float32)
    out_ref[...] = acc + bias_ref[...]


def kernel(x, coeffs, bias, knots):
    c2 = coeffs.reshape(_O, _N)
    grid = _B // _BT
    out = pl.pallas_call(
        _body,
        grid=(grid,),
        in_specs=[
            pl.BlockSpec((_BT, _I), lambda b: (b, 0)),
            pl.BlockSpec((_O, _N), lambda b: (0, 0)),
            pl.BlockSpec((1, _K), lambda b: (0, 0)),
            pl.BlockSpec((1, _O), lambda b: (0, 0)),
        ],
        out_specs=pl.BlockSpec((_BT, _O), lambda b: (b, 0)),
        out_shape=jax.ShapeDtypeStruct((_B, _O), jnp.float32),
        scratch_shapes=[pltpu.VMEM((_O, _N), jnp.float32)],
    )(x, c2, knots.reshape(1, _K), bias.reshape(1, _O))
    return out
